# baseline (speedup 1.0000x reference)
import functools

import jax
import jax.numpy as jnp
from jax import lax
from jax.experimental import pallas as pl
from jax.experimental.pallas import tpu as pltpu
from jax.experimental.pallas import tpu_sc as plsc

VOCAB = 100000
HIDDEN = 768
MAX_POS = 2048
B, S = 4, 2048
EPS = 1e-12

NC, NS, L = 2, 16, 16
NW = NC * NS
NTOK = B * S
SPW = S // NW
CH = 32
CPB = SPW // CH
NG = CH // L
PPR = HIDDEN // 128
PCH = CH * PPR
M16 = 128 // L
WP = VOCAB * PPR
OP = NTOK * PPR


def _rsqrt(x):
    i = plsc.bitcast(x, jnp.int32)
    i = jnp.int32(0x5F3759DF) - lax.shift_right_logical(i, jnp.int32(1))
    y = plsc.bitcast(i, jnp.float32)
    for _ in range(3):
        y = y * (1.5 - 0.5 * x * y * y)
    return y


def _as_pieces(a, rows):
    return (a.reshape(rows // 8, 8, PPR, 128)
            .transpose(0, 2, 1, 3)
            .reshape(rows * PPR, 128))


def _sc_embed(ids_flat, word_p, pos_p, type_emb, gamma, beta):
    mesh = plsc.VectorSubcoreMesh(core_axis_name="c", subcore_axis_name="s")

    @functools.partial(
        pl.kernel,
        mesh=mesh,
        compiler_params=pltpu.CompilerParams(use_tc_tiling_on_sc=False,
                                             needs_layout_passes=False),
        out_type=jax.ShapeDtypeStruct((OP, 128), jnp.float32),
        scratch_types=[
            pltpu.VMEM((NTOK // NW,), jnp.int32),
            pltpu.VMEM((PCH // 2,), jnp.int32),
            pltpu.VMEM((PCH // 2,), jnp.int32),
            pltpu.VMEM((PCH // 2,), jnp.int32),
            pltpu.VMEM((PCH // 2,), jnp.int32),
            pltpu.VMEM((PCH, 128), jnp.float32),
            pltpu.VMEM((PCH, 128), jnp.float32),
            pltpu.VMEM((SPW * PPR, 128), jnp.float32),
            pltpu.VMEM((HIDDEN,), jnp.float32),
            pltpu.VMEM((HIDDEN,), jnp.float32),
            pltpu.VMEM((HIDDEN,), jnp.float32),
            pltpu.VMEM((L, CH), jnp.float32),
            pltpu.VMEM((L, CH), jnp.float32),
            pltpu.VMEM((CH,), jnp.float32),
            pltpu.VMEM((CH,), jnp.float32),
            pltpu.SemaphoreType.DMA,
            pltpu.SemaphoreType.DMA,
            pltpu.SemaphoreType.DMA,
            pltpu.SemaphoreType.DMA,
        ],
    )
    def k(ids_hbm, word_hbm, pos_hbm, type_hbm, gamma_hbm, beta_hbm,
          out_hbm, ids_v, pidxa0, pidxb0, pidxa1, pidxb1, rows0, rows1,
          pos_v, type_v, gamma_v, beta_v, sums_v, sumsq_v, mean_v, scale_v,
          gsem0, gsem1, osem0, osem1):
        wid = lax.axis_index("s") * NC + lax.axis_index("c")
        s_base = wid * SPW
        rows = (rows0, rows1)
        gsem = (gsem0, gsem1)
        osem = (osem0, osem1)

        pltpu.sync_copy(type_hbm.at[0], type_v)
        pltpu.sync_copy(gamma_hbm, gamma_v)
        pltpu.sync_copy(beta_hbm, beta_v)
        pltpu.sync_copy(pos_hbm.at[pl.ds(s_base * PPR, SPW * PPR)], pos_v)

        lanes = lax.iota(jnp.int32, L)
        posv = (lanes >> 3) * 48 + (lanes & 7)

        pidx = ((pidxa0, pidxb0), (pidxa1, pidxb1))

        def fill_indices(b, coff):
            for h, ref in ((0, pidx[b][0]), (1, pidx[b][1])):
                ids_h = ids_v[pl.ds(coff + h * L, L)]
                base = (ids_h >> 3) * 48 + (ids_h & 7)
                for kk in range(PPR):
                    plsc.store_scatter(ref, [posv + 8 * kk], base + 8 * kk)

        def start_gather(b):
            pltpu.async_copy(word_hbm.at[pidx[b][0]],
                             rows[b].at[pl.ds(0, 96)], gsem[b])
            pltpu.async_copy(word_hbm.at[pidx[b][1]],
                             rows[b].at[pl.ds(96, 96)], gsem[b])

        def wait_gather(b):
            pltpu.make_async_copy(word_hbm.at[pidx[b][0]],
                                  rows[b].at[pl.ds(0, 96)], gsem[b]).wait()
            pltpu.make_async_copy(word_hbm.at[pidx[b][1]],
                                  rows[b].at[pl.ds(96, 96)], gsem[b]).wait()

        def start_out(b, tok0):
            pltpu.async_copy(rows[b], out_hbm.at[pl.ds(tok0 * PPR, PCH)],
                             osem[b])

        def wait_out(b, tok0):
            pltpu.make_async_copy(rows[b], out_hbm.at[pl.ds(tok0 * PPR, PCH)],
                                  osem[b]).wait()

        def compute_chunk(s_off, buf):
            poff = s_off * PPR

            for kk in range(PPR):
                tvs = [type_v[pl.ds(kk * 128 + m * L, L)] for m in range(M16)]

                @plsc.parallel_loop(0, CH, unroll=4)
                def p1(t):
                    pr = (t // 8) * 48 + lax.rem(t, 8) + 8 * kk
                    a0 = jnp.zeros((L,), jnp.float32)
                    a1 = jnp.zeros((L,), jnp.float32)
                    q0 = jnp.zeros((L,), jnp.float32)
                    q1 = jnp.zeros((L,), jnp.float32)
                    for m in range(M16):
                        sl = pl.ds(m * L, L)
                        v = buf[pr, sl] + pos_v[pr + poff, sl] + tvs[m]
                        buf[pr, sl] = v
                        if m % 2 == 0:
                            a0 = a0 + v
                            q0 = q0 + v * v
                        else:
                            a1 = a1 + v
                            q1 = q1 + v * v
                    colt = jnp.full((L,), t, jnp.int32)
                    if kk == 0:
                        plsc.store_scatter(sums_v, [lanes, colt], a0 + a1)
                        plsc.store_scatter(sumsq_v, [lanes, colt], q0 + q1)
                    else:
                        plsc.addupdate_scatter(sums_v, [lanes, colt], a0 + a1)
                        plsc.addupdate_scatter(sumsq_v, [lanes, colt],
                                               q0 + q1)

            for g in range(NG):
                sl = pl.ds(g * L, L)
                t0 = sums_v[0, sl]
                t1 = sums_v[1, sl]
                s0 = sumsq_v[0, sl]
                s1 = sumsq_v[1, sl]
                for l in range(2, L, 2):
                    t0 = t0 + sums_v[l, sl]
                    t1 = t1 + sums_v[l + 1, sl]
                    s0 = s0 + sumsq_v[l, sl]
                    s1 = s1 + sumsq_v[l + 1, sl]
                mean = (t0 + t1) * (1.0 / HIDDEN)
                var = (s0 + s1) * (1.0 / HIDDEN) - mean * mean
                var = jnp.maximum(var, 0.0)
                mean_v[sl] = mean
                scale_v[sl] = _rsqrt(var + EPS)

            for kk in range(PPR):
                gvs = [gamma_v[pl.ds(kk * 128 + m * L, L)] for m in range(M16)]
                bvs = [beta_v[pl.ds(kk * 128 + m * L, L)] for m in range(M16)]

                @plsc.parallel_loop(0, CH, unroll=4)
                def p2(t):
                    pr = (t // 8) * 48 + lax.rem(t, 8) + 8 * kk
                    tv = jnp.full((L,), t, jnp.int32)
                    mb = plsc.load_gather(mean_v, [tv])
                    sb = plsc.load_gather(scale_v, [tv])
                    for m in range(M16):
                        sl = pl.ds(m * L, L)
                        v = (buf[pr, sl] - mb) * sb
                        buf[pr, sl] = v * gvs[m] + bvs[m]

        for i in range(B):
            pltpu.sync_copy(ids_hbm.at[pl.ds(i * S + s_base, 2 * CH)],
                            ids_v.at[pl.ds(i * 2 * CH, 2 * CH)])

        fill_indices(0, 0)
        start_gather(0)
        fill_indices(1, CH)
        start_gather(1)

        def batch_body(i, carry):
            tok_a = i * S + s_base
            tok_b = tok_a + CH

            wait_gather(0)

            @pl.when(i < B - 1)
            def _():
                fill_indices(0, (i + 1) * 2 * CH)

            compute_chunk(0, rows[0])
            start_out(0, tok_a)

            wait_gather(1)

            @pl.when(i < B - 1)
            def _():
                fill_indices(1, (i + 1) * 2 * CH + CH)
                wait_out(0, tok_a)
                start_gather(0)

            compute_chunk(CH, rows[1])
            start_out(1, tok_b)

            @pl.when(i < B - 1)
            def _():
                wait_out(1, tok_b)
                start_gather(1)

            return carry

        lax.fori_loop(0, B, batch_body, 0)
        last_a = (B - 1) * S + s_base
        wait_out(0, last_a)
        wait_out(1, last_a + CH)

    return k(ids_flat, word_p, pos_p, type_emb, gamma, beta)


@jax.jit
def kernel(input_ids, word_emb, pos_emb, type_emb, gamma, beta):
    ids_flat = input_ids.reshape(NTOK).astype(jnp.int32)
    word_p = _as_pieces(word_emb, VOCAB)
    pos_p = _as_pieces(pos_emb, MAX_POS)
    out_p = _sc_embed(ids_flat, word_p, pos_p, type_emb, gamma, beta)
    out = (out_p.reshape(NTOK // 8, PPR, 8, 128)
           .transpose(0, 2, 1, 3)
           .reshape(NTOK, HIDDEN))
    return out.reshape(B, S, HIDDEN)


# speedup vs baseline: 1.0571x; 1.0571x over previous
import functools

import jax
import jax.numpy as jnp
from jax import lax
from jax.experimental import pallas as pl
from jax.experimental.pallas import tpu as pltpu
from jax.experimental.pallas import tpu_sc as plsc

VOCAB = 100000
HIDDEN = 768
MAX_POS = 2048
B, S = 4, 2048
EPS = 1e-12

NC, NS, L = 2, 16, 16
NW = NC * NS
NTOK = B * S
SPW = S // NW
CH = 32
CPB = SPW // CH
NG = CH // L
PPR = HIDDEN // 128
PCH = CH * PPR
M16 = 128 // L
WP = VOCAB * PPR
OP = NTOK * PPR


def _rsqrt(x):
    i = plsc.bitcast(x, jnp.int32)
    i = jnp.int32(0x5F3759DF) - lax.shift_right_logical(i, jnp.int32(1))
    y = plsc.bitcast(i, jnp.float32)
    for _ in range(3):
        y = y * (1.5 - 0.5 * x * y * y)
    return y


def _as_pieces(a, rows):
    return (a.reshape(rows // 8, 8, PPR, 128)
            .transpose(0, 2, 1, 3)
            .reshape(rows * PPR, 128))


def _sc_embed(ids_flat, word_p, pos_p, type_emb, gamma, beta):
    mesh = plsc.VectorSubcoreMesh(core_axis_name="c", subcore_axis_name="s")

    @functools.partial(
        pl.kernel,
        mesh=mesh,
        compiler_params=pltpu.CompilerParams(use_tc_tiling_on_sc=False,
                                             needs_layout_passes=False),
        out_type=jax.ShapeDtypeStruct((OP, 128), jnp.float32),
        scratch_types=[
            pltpu.VMEM((NTOK // NW,), jnp.int32),
            pltpu.VMEM((PCH // 2,), jnp.int32),
            pltpu.VMEM((PCH // 2,), jnp.int32),
            pltpu.VMEM((PCH // 2,), jnp.int32),
            pltpu.VMEM((PCH // 2,), jnp.int32),
            pltpu.VMEM((PCH, 128), jnp.float32),
            pltpu.VMEM((PCH, 128), jnp.float32),
            pltpu.VMEM((SPW * PPR, 128), jnp.float32),
            pltpu.VMEM((HIDDEN,), jnp.float32),
            pltpu.VMEM((HIDDEN,), jnp.float32),
            pltpu.VMEM((HIDDEN,), jnp.float32),
            pltpu.VMEM((L, CH), jnp.float32),
            pltpu.VMEM((L, CH), jnp.float32),
            pltpu.VMEM((CH,), jnp.float32),
            pltpu.VMEM((CH,), jnp.float32),
            pltpu.SemaphoreType.DMA,
            pltpu.SemaphoreType.DMA,
            pltpu.SemaphoreType.DMA,
            pltpu.SemaphoreType.DMA,
        ],
    )
    def k(ids_hbm, word_hbm, pos_hbm, type_hbm, gamma_hbm, beta_hbm,
          out_hbm, ids_v, pidxa0, pidxb0, pidxa1, pidxb1, rows0, rows1,
          pos_v, type_v, gamma_v, beta_v, sums_v, sumsq_v, mean_v, scale_v,
          gsem0, gsem1, osem0, osem1):
        wid = lax.axis_index("s") * NC + lax.axis_index("c")
        s_base = wid * SPW
        rows = (rows0, rows1)
        gsem = (gsem0, gsem1)
        osem = (osem0, osem1)

        pltpu.sync_copy(type_hbm.at[0], type_v)
        pltpu.sync_copy(gamma_hbm, gamma_v)
        pltpu.sync_copy(beta_hbm, beta_v)
        pltpu.sync_copy(pos_hbm.at[pl.ds(s_base * PPR, SPW * PPR)], pos_v)

        lanes = lax.iota(jnp.int32, L)
        posv = (lanes >> 3) * 48 + (lanes & 7)

        pidx = ((pidxa0, pidxb0), (pidxa1, pidxb1))

        def fill_indices(b, coff):
            for h, ref in ((0, pidx[b][0]), (1, pidx[b][1])):
                ids_h = ids_v[pl.ds(coff + h * L, L)]
                base = (ids_h >> 3) * 48 + (ids_h & 7)
                for kk in range(PPR):
                    plsc.store_scatter(ref, [posv + 8 * kk], base + 8 * kk)

        def start_gather(b):
            pltpu.async_copy(word_hbm.at[pidx[b][0]],
                             rows[b].at[pl.ds(0, 96)], gsem[b])
            pltpu.async_copy(word_hbm.at[pidx[b][1]],
                             rows[b].at[pl.ds(96, 96)], gsem[b])

        def wait_gather(b):
            pltpu.make_async_copy(word_hbm.at[pidx[b][0]],
                                  rows[b].at[pl.ds(0, 96)], gsem[b]).wait()
            pltpu.make_async_copy(word_hbm.at[pidx[b][1]],
                                  rows[b].at[pl.ds(96, 96)], gsem[b]).wait()

        def start_out(b, tok0):
            pltpu.async_copy(rows[b], out_hbm.at[pl.ds(tok0 * PPR, PCH)],
                             osem[b])

        def wait_out(b, tok0):
            pltpu.make_async_copy(rows[b], out_hbm.at[pl.ds(tok0 * PPR, PCH)],
                                  osem[b]).wait()

        def compute_chunk(s_off, buf):
            poff = s_off * PPR

            for kk in range(PPR):
                tvs = [type_v[pl.ds(kk * 128 + m * L, L)] for m in range(M16)]

                @plsc.parallel_loop(0, CH, unroll=2)
                def p1(t):
                    pr = (t // 8) * 48 + lax.rem(t, 8) + 8 * kk
                    a0 = jnp.zeros((L,), jnp.float32)
                    a1 = jnp.zeros((L,), jnp.float32)
                    q0 = jnp.zeros((L,), jnp.float32)
                    q1 = jnp.zeros((L,), jnp.float32)
                    for m in range(M16):
                        sl = pl.ds(m * L, L)
                        v = buf[pr, sl] + pos_v[pr + poff, sl] + tvs[m]
                        buf[pr, sl] = v
                        if m % 2 == 0:
                            a0 = a0 + v
                            q0 = q0 + v * v
                        else:
                            a1 = a1 + v
                            q1 = q1 + v * v
                    colt = jnp.full((L,), t, jnp.int32)
                    if kk == 0:
                        plsc.store_scatter(sums_v, [lanes, colt], a0 + a1)
                        plsc.store_scatter(sumsq_v, [lanes, colt], q0 + q1)
                    else:
                        plsc.addupdate_scatter(sums_v, [lanes, colt], a0 + a1)
                        plsc.addupdate_scatter(sumsq_v, [lanes, colt],
                                               q0 + q1)

            for g in range(NG):
                sl = pl.ds(g * L, L)
                t0 = sums_v[0, sl]
                t1 = sums_v[1, sl]
                s0 = sumsq_v[0, sl]
                s1 = sumsq_v[1, sl]
                for l in range(2, L, 2):
                    t0 = t0 + sums_v[l, sl]
                    t1 = t1 + sums_v[l + 1, sl]
                    s0 = s0 + sumsq_v[l, sl]
                    s1 = s1 + sumsq_v[l + 1, sl]
                mean = (t0 + t1) * (1.0 / HIDDEN)
                var = (s0 + s1) * (1.0 / HIDDEN) - mean * mean
                var = jnp.maximum(var, 0.0)
                mean_v[sl] = mean
                scale_v[sl] = _rsqrt(var + EPS)

            for kk in range(PPR):
                gvs = [gamma_v[pl.ds(kk * 128 + m * L, L)] for m in range(M16)]
                bvs = [beta_v[pl.ds(kk * 128 + m * L, L)] for m in range(M16)]

                @plsc.parallel_loop(0, CH, unroll=2)
                def p2(t):
                    pr = (t // 8) * 48 + lax.rem(t, 8) + 8 * kk
                    tv = jnp.full((L,), t, jnp.int32)
                    mb = plsc.load_gather(mean_v, [tv])
                    sb = plsc.load_gather(scale_v, [tv])
                    for m in range(M16):
                        sl = pl.ds(m * L, L)
                        v = (buf[pr, sl] - mb) * sb
                        buf[pr, sl] = v * gvs[m] + bvs[m]

        for i in range(B):
            pltpu.sync_copy(ids_hbm.at[pl.ds(i * S + s_base, 2 * CH)],
                            ids_v.at[pl.ds(i * 2 * CH, 2 * CH)])

        fill_indices(0, 0)
        start_gather(0)
        fill_indices(1, CH)
        start_gather(1)

        def batch_body(i, carry):
            tok_a = i * S + s_base
            tok_b = tok_a + CH

            wait_gather(0)

            @pl.when(i < B - 1)
            def _():
                fill_indices(0, (i + 1) * 2 * CH)

            compute_chunk(0, rows[0])
            start_out(0, tok_a)

            wait_gather(1)

            @pl.when(i < B - 1)
            def _():
                fill_indices(1, (i + 1) * 2 * CH + CH)
                wait_out(0, tok_a)
                start_gather(0)

            compute_chunk(CH, rows[1])
            start_out(1, tok_b)

            @pl.when(i < B - 1)
            def _():
                wait_out(1, tok_b)
                start_gather(1)

            return carry

        lax.fori_loop(0, B, batch_body, 0)
        last_a = (B - 1) * S + s_base
        wait_out(0, last_a)
        wait_out(1, last_a + CH)

    return k(ids_flat, word_p, pos_p, type_emb, gamma, beta)


@jax.jit
def kernel(input_ids, word_emb, pos_emb, type_emb, gamma, beta):
    ids_flat = input_ids.reshape(NTOK).astype(jnp.int32)
    word_p = _as_pieces(word_emb, VOCAB)
    pos_p = _as_pieces(pos_emb, MAX_POS)
    out_p = _sc_embed(ids_flat, word_p, pos_p, type_emb, gamma, beta)
    out = (out_p.reshape(NTOK // 8, PPR, 8, 128)
           .transpose(0, 2, 1, 3)
           .reshape(NTOK, HIDDEN))
    return out.reshape(B, S, HIDDEN)


# speedup vs baseline: 1.0724x; 1.0144x over previous
import functools

import jax
import jax.numpy as jnp
from jax import lax
from jax.experimental import pallas as pl
from jax.experimental.pallas import tpu as pltpu
from jax.experimental.pallas import tpu_sc as plsc

VOCAB = 100000
HIDDEN = 768
MAX_POS = 2048
B, S = 4, 2048
EPS = 1e-12

NC, NS, L = 2, 16, 16
NW = NC * NS
NTOK = B * S
SPW = S // NW
CH = 32
CPB = SPW // CH
NG = CH // L
PPR = HIDDEN // 128
PCH = CH * PPR
M16 = 128 // L
WP = VOCAB * PPR
OP = NTOK * PPR


def _rsqrt(x):
    i = plsc.bitcast(x, jnp.int32)
    i = jnp.int32(0x5F3759DF) - lax.shift_right_logical(i, jnp.int32(1))
    y = plsc.bitcast(i, jnp.float32)
    for _ in range(3):
        y = y * (1.5 - 0.5 * x * y * y)
    return y


def _as_pieces(a, rows):
    return (a.reshape(rows // 8, 8, PPR, 128)
            .transpose(0, 2, 1, 3)
            .reshape(rows * PPR, 128))


def _sc_embed(ids_flat, word_p, pos_p, type_emb, gamma, beta):
    mesh = plsc.VectorSubcoreMesh(core_axis_name="c", subcore_axis_name="s")

    @functools.partial(
        pl.kernel,
        mesh=mesh,
        compiler_params=pltpu.CompilerParams(use_tc_tiling_on_sc=False,
                                             needs_layout_passes=False),
        out_type=jax.ShapeDtypeStruct((OP, 128), jnp.float32),
        scratch_types=[
            pltpu.VMEM((NTOK // NW,), jnp.int32),
            pltpu.VMEM((PCH // 2,), jnp.int32),
            pltpu.VMEM((PCH // 2,), jnp.int32),
            pltpu.VMEM((PCH // 2,), jnp.int32),
            pltpu.VMEM((PCH // 2,), jnp.int32),
            pltpu.VMEM((PCH, 128), jnp.float32),
            pltpu.VMEM((PCH, 128), jnp.float32),
            pltpu.VMEM((SPW * PPR, 128), jnp.float32),
            pltpu.VMEM((HIDDEN,), jnp.float32),
            pltpu.VMEM((HIDDEN,), jnp.float32),
            pltpu.VMEM((HIDDEN,), jnp.float32),
            pltpu.VMEM((L, CH), jnp.float32),
            pltpu.VMEM((L, CH), jnp.float32),
            pltpu.VMEM((CH,), jnp.float32),
            pltpu.VMEM((CH,), jnp.float32),
            pltpu.SemaphoreType.DMA,
            pltpu.SemaphoreType.DMA,
            pltpu.SemaphoreType.DMA,
            pltpu.SemaphoreType.DMA,
        ],
    )
    def k(ids_hbm, word_hbm, pos_hbm, type_hbm, gamma_hbm, beta_hbm,
          out_hbm, ids_v, pidxa0, pidxb0, pidxa1, pidxb1, rows0, rows1,
          pos_v, type_v, gamma_v, beta_v, sums_v, sumsq_v, mean_v, scale_v,
          gsem0, gsem1, osem0, osem1):
        wid = lax.axis_index("s") * NC + lax.axis_index("c")
        s_base = wid * SPW
        rows = (rows0, rows1)
        gsem = (gsem0, gsem1)
        osem = (osem0, osem1)

        pltpu.sync_copy(type_hbm.at[0], type_v)
        pltpu.sync_copy(gamma_hbm, gamma_v)
        pltpu.sync_copy(beta_hbm, beta_v)
        pltpu.sync_copy(pos_hbm.at[pl.ds(s_base * PPR, SPW * PPR)], pos_v)

        lanes = lax.iota(jnp.int32, L)
        posv = (lanes >> 3) * 48 + (lanes & 7)

        pidx = ((pidxa0, pidxb0), (pidxa1, pidxb1))

        def fill_indices(b, coff):
            for h, ref in ((0, pidx[b][0]), (1, pidx[b][1])):
                ids_h = ids_v[pl.ds(coff + h * L, L)]
                base = (ids_h >> 3) * 48 + (ids_h & 7)
                for kk in range(PPR):
                    plsc.store_scatter(ref, [posv + 8 * kk], base + 8 * kk)

        def start_gather(b):
            pltpu.async_copy(word_hbm.at[pidx[b][0]],
                             rows[b].at[pl.ds(0, 96)], gsem[b])
            pltpu.async_copy(word_hbm.at[pidx[b][1]],
                             rows[b].at[pl.ds(96, 96)], gsem[b])

        def wait_gather(b):
            pltpu.make_async_copy(word_hbm.at[pidx[b][0]],
                                  rows[b].at[pl.ds(0, 96)], gsem[b]).wait()
            pltpu.make_async_copy(word_hbm.at[pidx[b][1]],
                                  rows[b].at[pl.ds(96, 96)], gsem[b]).wait()

        def start_out(b, tok0):
            pltpu.async_copy(rows[b], out_hbm.at[pl.ds(tok0 * PPR, PCH)],
                             osem[b])

        def wait_out(b, tok0):
            pltpu.make_async_copy(rows[b], out_hbm.at[pl.ds(tok0 * PPR, PCH)],
                                  osem[b]).wait()

        def compute_chunk(s_off, buf):
            poff = s_off * PPR

            for kk in range(PPR):
                tvs = [type_v[pl.ds(kk * 128 + m * L, L)] for m in range(M16)]

                @plsc.parallel_loop(0, CH, unroll=2)
                def p1(t):
                    pr = (t // 8) * 48 + lax.rem(t, 8) + 8 * kk
                    a0 = jnp.zeros((L,), jnp.float32)
                    a1 = jnp.zeros((L,), jnp.float32)
                    q0 = jnp.zeros((L,), jnp.float32)
                    q1 = jnp.zeros((L,), jnp.float32)
                    for m in range(M16):
                        sl = pl.ds(m * L, L)
                        v = buf[pr, sl] + pos_v[pr + poff, sl] + tvs[m]
                        buf[pr, sl] = v
                        if m % 2 == 0:
                            a0 = a0 + v
                            q0 = q0 + v * v
                        else:
                            a1 = a1 + v
                            q1 = q1 + v * v
                    colt = jnp.full((L,), t, jnp.int32)
                    if kk == 0:
                        plsc.store_scatter(sums_v, [lanes, colt], a0 + a1)
                        plsc.store_scatter(sumsq_v, [lanes, colt], q0 + q1)
                    else:
                        plsc.addupdate_scatter(sums_v, [lanes, colt], a0 + a1)
                        plsc.addupdate_scatter(sumsq_v, [lanes, colt],
                                               q0 + q1)

            for g in range(NG):
                sl = pl.ds(g * L, L)
                t0 = sums_v[0, sl]
                t1 = sums_v[1, sl]
                s0 = sumsq_v[0, sl]
                s1 = sumsq_v[1, sl]
                for l in range(2, L, 2):
                    t0 = t0 + sums_v[l, sl]
                    t1 = t1 + sums_v[l + 1, sl]
                    s0 = s0 + sumsq_v[l, sl]
                    s1 = s1 + sumsq_v[l + 1, sl]
                mean = (t0 + t1) * (1.0 / HIDDEN)
                var = (s0 + s1) * (1.0 / HIDDEN) - mean * mean
                var = jnp.maximum(var, 0.0)
                mean_v[sl] = mean
                scale_v[sl] = _rsqrt(var + EPS)

            for kk in range(PPR):
                gvs = [gamma_v[pl.ds(kk * 128 + m * L, L)] for m in range(M16)]
                bvs = [beta_v[pl.ds(kk * 128 + m * L, L)] for m in range(M16)]

                @plsc.parallel_loop(0, CH, unroll=2)
                def p2(t):
                    pr = (t // 8) * 48 + lax.rem(t, 8) + 8 * kk
                    tv = jnp.full((L,), t, jnp.int32)
                    mb = plsc.load_gather(mean_v, [tv])
                    sb = plsc.load_gather(scale_v, [tv])
                    for m in range(M16):
                        sl = pl.ds(m * L, L)
                        v = (buf[pr, sl] - mb) * sb
                        buf[pr, sl] = v * gvs[m] + bvs[m]

        for i in range(B):
            pltpu.sync_copy(ids_hbm.at[pl.ds(i * S + s_base, 2 * CH)],
                            ids_v.at[pl.ds(i * 2 * CH, 2 * CH)])

        fill_indices(0, 0)
        start_gather(0)

        def batch_body(i, carry):
            tok_a = i * S + s_base
            tok_b = tok_a + CH

            wait_gather(0)
            fill_indices(1, i * 2 * CH + CH)

            @pl.when(i >= 1)
            def _():
                wait_out(1, tok_b)

            start_gather(1)
            compute_chunk(0, rows[0])
            start_out(0, tok_a)

            wait_gather(1)

            @pl.when(i < B - 1)
            def _():
                fill_indices(0, (i + 1) * 2 * CH)
                wait_out(0, tok_a)
                start_gather(0)

            compute_chunk(CH, rows[1])
            start_out(1, tok_b)
            return carry

        lax.fori_loop(0, B, batch_body, 0)
        last_a = (B - 1) * S + s_base
        wait_out(0, last_a)
        wait_out(1, last_a + CH)

    return k(ids_flat, word_p, pos_p, type_emb, gamma, beta)


@jax.jit
def kernel(input_ids, word_emb, pos_emb, type_emb, gamma, beta):
    ids_flat = input_ids.reshape(NTOK).astype(jnp.int32)
    word_p = _as_pieces(word_emb, VOCAB)
    pos_p = _as_pieces(pos_emb, MAX_POS)
    out_p = _sc_embed(ids_flat, word_p, pos_p, type_emb, gamma, beta)
    out = (out_p.reshape(NTOK // 8, PPR, 8, 128)
           .transpose(0, 2, 1, 3)
           .reshape(NTOK, HIDDEN))
    return out.reshape(B, S, HIDDEN)


# speedup vs baseline: 1.0882x; 1.0148x over previous
import functools

import jax
import jax.numpy as jnp
from jax import lax
from jax.experimental import pallas as pl
from jax.experimental.pallas import tpu as pltpu
from jax.experimental.pallas import tpu_sc as plsc

VOCAB = 100000
HIDDEN = 768
MAX_POS = 2048
B, S = 4, 2048
EPS = 1e-12

NC, NS, L = 2, 16, 16
NW = NC * NS
NTOK = B * S
SPW = S // NW
CH = 32
CPB = SPW // CH
NG = CH // L
PPR = HIDDEN // 128
PCH = CH * PPR
M16 = 128 // L
WP = VOCAB * PPR
OP = NTOK * PPR


def _rsqrt(x):
    i = plsc.bitcast(x, jnp.int32)
    i = jnp.int32(0x5F3759DF) - lax.shift_right_logical(i, jnp.int32(1))
    y = plsc.bitcast(i, jnp.float32)
    for _ in range(3):
        y = y * (1.5 - 0.5 * x * y * y)
    return y


def _as_pieces(a, rows):
    return (a.reshape(rows // 8, 8, PPR, 128)
            .transpose(0, 2, 1, 3)
            .reshape(rows * PPR, 128))


def _sc_embed(ids_flat, word_p, pos_p, type_emb, gamma, beta):
    mesh = plsc.VectorSubcoreMesh(core_axis_name="c", subcore_axis_name="s")

    @functools.partial(
        pl.kernel,
        mesh=mesh,
        compiler_params=pltpu.CompilerParams(use_tc_tiling_on_sc=False,
                                             needs_layout_passes=False),
        out_type=jax.ShapeDtypeStruct((OP, 128), jnp.float32),
        scratch_types=[
            pltpu.VMEM((NTOK // NW,), jnp.int32),
            pltpu.VMEM((PCH // 2,), jnp.int32),
            pltpu.VMEM((PCH // 2,), jnp.int32),
            pltpu.VMEM((PCH // 2,), jnp.int32),
            pltpu.VMEM((PCH // 2,), jnp.int32),
            pltpu.VMEM((PCH, 128), jnp.float32),
            pltpu.VMEM((PCH, 128), jnp.float32),
            pltpu.VMEM((SPW * PPR, 128), jnp.float32),
            pltpu.VMEM((HIDDEN,), jnp.float32),
            pltpu.VMEM((HIDDEN,), jnp.float32),
            pltpu.VMEM((HIDDEN,), jnp.float32),
            pltpu.VMEM((L, CH), jnp.float32),
            pltpu.VMEM((L, CH), jnp.float32),
            pltpu.VMEM((CH,), jnp.float32),
            pltpu.VMEM((CH,), jnp.float32),
            pltpu.SemaphoreType.DMA,
            pltpu.SemaphoreType.DMA,
            pltpu.SemaphoreType.DMA,
            pltpu.SemaphoreType.DMA,
        ],
    )
    def k(ids_hbm, word_hbm, pos_hbm, type_hbm, gamma_hbm, beta_hbm,
          out_hbm, ids_v, pidxa0, pidxb0, pidxa1, pidxb1, rows0, rows1,
          pos_v, type_v, gamma_v, beta_v, sums_v, sumsq_v, mean_v, scale_v,
          gsem0, gsem1, osem0, osem1):
        wid = lax.axis_index("s") * NC + lax.axis_index("c")
        s_base = wid * SPW
        rows = (rows0, rows1)
        gsem = (gsem0, gsem1)
        osem = (osem0, osem1)

        lanes = lax.iota(jnp.int32, L)
        posv = (lanes >> 3) * 48 + (lanes & 7)

        pidx = ((pidxa0, pidxb0), (pidxa1, pidxb1))

        def fill_indices(b, coff):
            for h, ref in ((0, pidx[b][0]), (1, pidx[b][1])):
                ids_h = ids_v[pl.ds(coff + h * L, L)]
                base = (ids_h >> 3) * 48 + (ids_h & 7)
                for kk in range(PPR):
                    plsc.store_scatter(ref, [posv + 8 * kk], base + 8 * kk)

        def start_gather(b):
            pltpu.async_copy(word_hbm.at[pidx[b][0]],
                             rows[b].at[pl.ds(0, 96)], gsem[b])
            pltpu.async_copy(word_hbm.at[pidx[b][1]],
                             rows[b].at[pl.ds(96, 96)], gsem[b])

        def wait_gather(b):
            pltpu.make_async_copy(word_hbm.at[pidx[b][0]],
                                  rows[b].at[pl.ds(0, 96)], gsem[b]).wait()
            pltpu.make_async_copy(word_hbm.at[pidx[b][1]],
                                  rows[b].at[pl.ds(96, 96)], gsem[b]).wait()

        def start_out(b, tok0):
            pltpu.async_copy(rows[b], out_hbm.at[pl.ds(tok0 * PPR, PCH)],
                             osem[b])

        def wait_out(b, tok0):
            pltpu.make_async_copy(rows[b], out_hbm.at[pl.ds(tok0 * PPR, PCH)],
                                  osem[b]).wait()

        def compute_chunk(s_off, buf):
            poff = s_off * PPR

            for kk in range(PPR):
                tvs = [type_v[pl.ds(kk * 128 + m * L, L)] for m in range(M16)]

                @plsc.parallel_loop(0, CH, unroll=2)
                def p1(t):
                    pr = (t // 8) * 48 + lax.rem(t, 8) + 8 * kk
                    a0 = jnp.zeros((L,), jnp.float32)
                    a1 = jnp.zeros((L,), jnp.float32)
                    q0 = jnp.zeros((L,), jnp.float32)
                    q1 = jnp.zeros((L,), jnp.float32)
                    for m in range(M16):
                        sl = pl.ds(m * L, L)
                        v = buf[pr, sl] + pos_v[pr + poff, sl] + tvs[m]
                        buf[pr, sl] = v
                        if m % 2 == 0:
                            a0 = a0 + v
                            q0 = q0 + v * v
                        else:
                            a1 = a1 + v
                            q1 = q1 + v * v
                    colt = jnp.full((L,), t, jnp.int32)
                    if kk == 0:
                        plsc.store_scatter(sums_v, [lanes, colt], a0 + a1)
                        plsc.store_scatter(sumsq_v, [lanes, colt], q0 + q1)
                    else:
                        plsc.addupdate_scatter(sums_v, [lanes, colt], a0 + a1)
                        plsc.addupdate_scatter(sumsq_v, [lanes, colt],
                                               q0 + q1)

            for g in range(NG):
                sl = pl.ds(g * L, L)
                t0 = sums_v[0, sl]
                t1 = sums_v[1, sl]
                s0 = sumsq_v[0, sl]
                s1 = sumsq_v[1, sl]
                for l in range(2, L, 2):
                    t0 = t0 + sums_v[l, sl]
                    t1 = t1 + sums_v[l + 1, sl]
                    s0 = s0 + sumsq_v[l, sl]
                    s1 = s1 + sumsq_v[l + 1, sl]
                mean = (t0 + t1) * (1.0 / HIDDEN)
                var = (s0 + s1) * (1.0 / HIDDEN) - mean * mean
                var = jnp.maximum(var, 0.0)
                mean_v[sl] = mean
                scale_v[sl] = _rsqrt(var + EPS)

            for kk in range(PPR):
                gvs = [gamma_v[pl.ds(kk * 128 + m * L, L)] for m in range(M16)]
                bvs = [beta_v[pl.ds(kk * 128 + m * L, L)] for m in range(M16)]

                @plsc.parallel_loop(0, CH, unroll=2)
                def p2(t):
                    pr = (t // 8) * 48 + lax.rem(t, 8) + 8 * kk
                    tv = jnp.full((L,), t, jnp.int32)
                    mb = plsc.load_gather(mean_v, [tv])
                    sb = plsc.load_gather(scale_v, [tv])
                    for m in range(M16):
                        sl = pl.ds(m * L, L)
                        v = (buf[pr, sl] - mb) * sb
                        buf[pr, sl] = v * gvs[m] + bvs[m]

        pltpu.sync_copy(ids_hbm.at[pl.ds(s_base, 2 * CH)],
                        ids_v.at[pl.ds(0, 2 * CH)])
        fill_indices(0, 0)
        start_gather(0)
        for i in range(1, B):
            pltpu.sync_copy(ids_hbm.at[pl.ds(i * S + s_base, 2 * CH)],
                            ids_v.at[pl.ds(i * 2 * CH, 2 * CH)])
        pltpu.sync_copy(type_hbm.at[0], type_v)
        pltpu.sync_copy(gamma_hbm, gamma_v)
        pltpu.sync_copy(beta_hbm, beta_v)
        pltpu.sync_copy(pos_hbm.at[pl.ds(s_base * PPR, SPW * PPR)], pos_v)

        def batch_body(i, carry):
            tok_a = i * S + s_base
            tok_b = tok_a + CH

            wait_gather(0)
            fill_indices(1, i * 2 * CH + CH)

            @pl.when(i >= 1)
            def _():
                wait_out(1, tok_b)

            start_gather(1)
            compute_chunk(0, rows[0])
            start_out(0, tok_a)

            wait_gather(1)

            @pl.when(i < B - 1)
            def _():
                fill_indices(0, (i + 1) * 2 * CH)
                wait_out(0, tok_a)
                start_gather(0)

            compute_chunk(CH, rows[1])
            start_out(1, tok_b)
            return carry

        lax.fori_loop(0, B, batch_body, 0)
        last_a = (B - 1) * S + s_base
        wait_out(0, last_a)
        wait_out(1, last_a + CH)

    return k(ids_flat, word_p, pos_p, type_emb, gamma, beta)


@jax.jit
def kernel(input_ids, word_emb, pos_emb, type_emb, gamma, beta):
    ids_flat = input_ids.reshape(NTOK).astype(jnp.int32)
    word_p = _as_pieces(word_emb, VOCAB)
    pos_p = _as_pieces(pos_emb, MAX_POS)
    out_p = _sc_embed(ids_flat, word_p, pos_p, type_emb, gamma, beta)
    out = (out_p.reshape(NTOK // 8, PPR, 8, 128)
           .transpose(0, 2, 1, 3)
           .reshape(NTOK, HIDDEN))
    return out.reshape(B, S, HIDDEN)
